# baseline (speedup 1.0000x reference)
import functools

import jax
import jax.numpy as jnp
from jax import lax
from jax.experimental import pallas as pl
from jax.experimental.pallas import tpu as pltpu

IN_PLANES = 64
OUT_PLANES = 64
STRIDE = 4
NSAMPLE = 16
EPS = 1e-5

_BIG_I32 = 2**30
_INF_F32 = 3e38


def _fps_body(px_ref, py_ref, pz_ref, feats_ref, wt_ref,
              fidx_ref, npx_ref, npy_ref, npz_ref, u_ref, *, seg, n, m, C):
    R = n // 128
    QR = m // 128
    px = px_ref[...]
    py = py_ref[...]
    pz = pz_ref[...]

    u_ref[...] = jnp.dot(
        feats_ref[...].reshape(seg * n, 128), wt_ref[...],
        preferred_element_type=jnp.float32).reshape(seg, n, 128)

    flat = (lax.broadcasted_iota(jnp.int32, (R, 128), 0) * 128
            + lax.broadcasted_iota(jnp.int32, (R, 128), 1))
    qflat = (lax.broadcasted_iota(jnp.int32, (QR, 128), 0) * 128
             + lax.broadcasted_iota(jnp.int32, (QR, 128), 1))

    dist_i = tuple(jnp.full((R, 128), 1e10, dtype=jnp.float32)
                   for _ in range(seg))
    l_i = tuple(pref[s, 0, 0]
                for s in range(seg) for pref in (px_ref, py_ref, pz_ref))
    idx_i = tuple(jnp.zeros((QR, 128), dtype=jnp.int32) for _ in range(seg))
    zero_q = jnp.zeros((QR, 128), dtype=jnp.float32)
    sel0 = qflat == 0
    np_i = tuple(jnp.where(sel0, l_i[3 * s + c], zero_q)
                 for s in range(seg) for c in range(3))

    lane1 = lax.broadcasted_iota(jnp.int32, (1, 128), 1)

    def body(i, st):
        dists = st[0:seg]
        ls = st[seg:4 * seg]
        idxs = list(st[4 * seg:5 * seg])
        nps = list(st[5 * seg:8 * seg])
        q = qflat == i
        ndists, nls = [], []
        for s in range(seg):
            lx, ly, lz = ls[3 * s:3 * s + 3]
            dx = px[s] - lx
            dy = py[s] - ly
            dz = pz[s] - lz
            d = dx * dx + dy * dy + dz * dz
            dist = jnp.minimum(dists[s], d)
            mx = jnp.max(dist)
            sel = jnp.min(jnp.where(dist == mx, flat, _BIG_I32))
            r = sel >> 7
            c = sel & 127
            lhit = lane1 == c
            sx = jnp.sum(jnp.where(lhit, px_ref[s, pl.ds(r, 1), :], 0.0))
            sy = jnp.sum(jnp.where(lhit, py_ref[s, pl.ds(r, 1), :], 0.0))
            sz = jnp.sum(jnp.where(lhit, pz_ref[s, pl.ds(r, 1), :], 0.0))
            ndists.append(dist)
            nls.extend((sx, sy, sz))
            idxs[s] = jnp.where(q, sel, idxs[s])
            nps[3 * s] = jnp.where(q, sx, nps[3 * s])
            nps[3 * s + 1] = jnp.where(q, sy, nps[3 * s + 1])
            nps[3 * s + 2] = jnp.where(q, sz, nps[3 * s + 2])
        return tuple(ndists) + tuple(nls) + tuple(idxs) + tuple(nps)

    st = lax.fori_loop(1, m, body, dist_i + l_i + idx_i + np_i)
    idxs = st[4 * seg:5 * seg]
    nps = st[5 * seg:8 * seg]
    for s in range(seg):
        fidx_ref[s] = idxs[s]
        npx_ref[s] = nps[3 * s]
        npy_ref[s] = nps[3 * s + 1]
        npz_ref[s] = nps[3 * s + 2]


def _run_fps(px, py, pz, feats, wt, *, seg, n, m, C):
    R = n // 128
    QR = m // 128
    body = functools.partial(_fps_body, seg=seg, n=n, m=m, C=C)
    out_shapes = (
        jax.ShapeDtypeStruct((seg, QR, 128), jnp.int32),
        jax.ShapeDtypeStruct((seg, QR, 128), jnp.float32),
        jax.ShapeDtypeStruct((seg, QR, 128), jnp.float32),
        jax.ShapeDtypeStruct((seg, QR, 128), jnp.float32),
        jax.ShapeDtypeStruct((seg, n, 128), jnp.float32),
    )
    full = lambda shape: pl.BlockSpec(shape, lambda: tuple(0 for _ in shape))
    return pl.pallas_call(
        body,
        in_specs=[
            full((seg, R, 128)), full((seg, R, 128)), full((seg, R, 128)),
            full((seg, n, 128)), full((128, 128)),
        ],
        out_specs=(
            full((seg, QR, 128)), full((seg, QR, 128)), full((seg, QR, 128)),
            full((seg, QR, 128)), full((seg, n, 128)),
        ),
        out_shape=out_shapes,
    )(px, py, pz, feats, wt)


_NPRE = 4
_CHW = 64


def _knn_body(pxb_ref, pyb_ref, pzb_ref, qx_ref, qy_ref, qz_ref,
              knn_ref, *, n, QB):
    seg = pl.program_id(0)
    NCH = n // _CHW
    px = pxb_ref[0][None]
    py = pyb_ref[0][None]
    pz = pzb_ref[0][None]
    qx = qx_ref[0][:, :, None]
    qy = qy_ref[0][:, :, None]
    qz = qz_ref[0][:, :, None]
    dx = qx - px
    dy = qy - py
    dz = qz - pz
    d2 = dx * dx + dy * dy + dz * dz

    row = lax.broadcasted_iota(jnp.int32, (QB, _CHW, NCH), 1)
    chunk64 = lax.broadcasted_iota(jnp.int32, (QB, 1, NCH), 2) * _CHW
    cvs, cis = [], []
    for _ in range(_NPRE):
        mv = jnp.min(d2, axis=1, keepdims=True)
        am = jnp.min(jnp.where(d2 == mv, row, _BIG_I32),
                     axis=1, keepdims=True)
        cvs.append(mv[:, 0, :])
        cis.append((chunk64 + am)[:, 0, :])
        d2 = jnp.where(row == am, _INF_F32, d2)
    cv = jnp.concatenate(cvs, axis=1)
    ci = jnp.concatenate(cis, axis=1)

    laneNS = lax.broadcasted_iota(jnp.int32, (QB, NSAMPLE), 1)
    knn = jnp.zeros((QB, NSAMPLE), dtype=jnp.int32)
    for t in range(NSAMPLE):
        mv = jnp.min(cv, axis=1, keepdims=True)
        am = jnp.min(jnp.where(cv == mv, ci, _BIG_I32),
                     axis=1, keepdims=True)
        knn = jnp.where(laneNS == t, am, knn)
        cv = jnp.where((cv == mv) & (ci == am), _INF_F32, cv)
    knn_ref[0] = knn + seg * n


def _run_knn(pxb, pyb, pzb, qx, qy, qz, *, seg, n, m, QB):
    nqb = m // QB
    NCH = n // _CHW
    body = functools.partial(_knn_body, n=n, QB=QB)
    pspec = pl.BlockSpec((1, _CHW, NCH), lambda s, q: (s, 0, 0))
    qspec = pl.BlockSpec((1, QB, 1), lambda s, q: (s, q, 0))
    return pl.pallas_call(
        body,
        grid=(seg, nqb),
        in_specs=[pspec, pspec, pspec, qspec, qspec, qspec],
        out_specs=pl.BlockSpec((1, QB, NSAMPLE), lambda s, q: (s, q, 0)),
        out_shape=jax.ShapeDtypeStruct((seg, m, NSAMPLE), jnp.int32),
    )(pxb, pyb, pzb, qx, qy, qz)


def _gather_body(u_ref, knn_ref, gsum_ref, gsq_ref, gmx_ref, gmn_ref, *, QB, C):
    def body(q, carry):
        i0 = knn_ref[q, 0]
        r = u_ref[pl.ds(i0, 1), :]
        s_acc = r
        q_acc = r * r
        mx = r
        mn = r
        for t in range(1, NSAMPLE):
            i = knn_ref[q, t]
            r = u_ref[pl.ds(i, 1), :]
            s_acc = s_acc + r
            q_acc = q_acc + r * r
            mx = jnp.maximum(mx, r)
            mn = jnp.minimum(mn, r)
        gsum_ref[pl.ds(q, 1), :] = s_acc
        gsq_ref[pl.ds(q, 1), :] = q_acc
        gmx_ref[pl.ds(q, 1), :] = mx
        gmn_ref[pl.ds(q, 1), :] = mn
        return carry

    lax.fori_loop(0, QB, body, 0)


def _run_gather_tc(u_all, knn_flat, *, M, Ntot, C, QB):
    nb = M // QB
    body = functools.partial(_gather_body, QB=QB, C=C)
    out_sp = pl.BlockSpec((QB, C), lambda b: (b, 0))
    return pl.pallas_call(
        body,
        grid=(nb,),
        in_specs=[
            pl.BlockSpec((Ntot, C), lambda b: (0, 0)),
            pl.BlockSpec((QB, NSAMPLE), lambda b: (b, 0),
                         memory_space=pltpu.SMEM),
        ],
        out_specs=(out_sp, out_sp, out_sp, out_sp),
        out_shape=tuple(jax.ShapeDtypeStruct((M, C), jnp.float32)
                        for _ in range(4)),
    )(u_all, knn_flat)


def _run_gather_sc(u_all, knn1d, *, M, Ntot, C):
    import jax.experimental.pallas.tpu_sc as plsc
    info = plsc.get_sparse_core_info()
    NC, NS_SC = info.num_cores, info.num_subcores
    NW = NC * NS_SC
    qpw = M // NW
    CQ = 16
    nch = qpw // CQ
    ROWS = CQ * NSAMPLE
    mesh = plsc.VectorSubcoreMesh(core_axis_name="c", subcore_axis_name="s")

    @functools.partial(
        pl.kernel, mesh=mesh,
        out_type=tuple(jax.ShapeDtypeStruct((M, C), jnp.float32)
                       for _ in range(4)),
        scratch_types=[
            pltpu.VMEM((ROWS,), jnp.int32),
            pltpu.VMEM((ROWS, 128), jnp.float32),
            pltpu.VMEM((CQ, C), jnp.float32),
            pltpu.VMEM((CQ, C), jnp.float32),
            pltpu.VMEM((CQ, C), jnp.float32),
            pltpu.VMEM((CQ, C), jnp.float32),
            pltpu.SemaphoreType.DMA,
        ],
    )
    def body(u_hbm, knn_hbm, gsum_hbm, gsq_hbm, gmx_hbm, gmn_hbm,
             idx_v, rows_v, st_sum, st_sq, st_mx, st_mn, sem):
        wid = lax.axis_index("s") * NC + lax.axis_index("c")
        qbase0 = wid * qpw

        def chunk(t, carry):
            qbase = qbase0 + t * CQ
            pltpu.sync_copy(knn_hbm.at[pl.ds(qbase * NSAMPLE, ROWS)], idx_v)
            pltpu.async_copy(u_hbm.at[idx_v], rows_v, sem).wait()

            def per_q(q, c2):
                b = q * NSAMPLE
                for v in range(C // 16):
                    sl = pl.ds(v * 16, 16)
                    r = rows_v[b, sl]
                    ssum = r
                    ssq = r * r
                    smx = r
                    smn = r
                    for s in range(1, NSAMPLE):
                        r = rows_v[b + s, sl]
                        ssum = ssum + r
                        ssq = ssq + r * r
                        smx = jnp.maximum(smx, r)
                        smn = jnp.minimum(smn, r)
                    st_sum[q, sl] = ssum
                    st_sq[q, sl] = ssq
                    st_mx[q, sl] = smx
                    st_mn[q, sl] = smn
                return c2

            lax.fori_loop(0, CQ, per_q, 0)
            pltpu.sync_copy(st_sum, gsum_hbm.at[pl.ds(qbase, CQ)])
            pltpu.sync_copy(st_sq, gsq_hbm.at[pl.ds(qbase, CQ)])
            pltpu.sync_copy(st_mx, gmx_hbm.at[pl.ds(qbase, CQ)])
            pltpu.sync_copy(st_mn, gmn_hbm.at[pl.ds(qbase, CQ)])
            return carry

        lax.fori_loop(0, nch, chunk, 0)

    return body(u_all, knn1d)


def _final_body(np_ref, wp_ref, gsum_ref, gsq_ref, gmx_ref, gmn_ref,
                g_ref, b_ref, out_ref, *, M, C):
    zq = jnp.dot(np_ref[...], wp_ref[...], preferred_element_type=jnp.float32)
    gsum = gsum_ref[...]
    cnt = jnp.float32(M * NSAMPLE)
    s1 = (jnp.sum(gsum, axis=0, keepdims=True)
          - NSAMPLE * jnp.sum(zq, axis=0, keepdims=True))
    s2 = (jnp.sum(gsq_ref[...], axis=0, keepdims=True)
          - 2.0 * jnp.sum(zq * gsum, axis=0, keepdims=True)
          + NSAMPLE * jnp.sum(zq * zq, axis=0, keepdims=True))
    mean = s1 / cnt
    var = s2 / cnt - mean * mean
    a = g_ref[...] * lax.rsqrt(var + EPS)
    hsel = jnp.where(a >= 0.0, gmx_ref[...], gmn_ref[...]) - zq
    out_ref[...] = jnp.maximum(a * (hsel - mean) + b_ref[...], 0.0)


def _run_final(np_pad, wp_pad, gsum, gsq, gmx, gmn, g2, b2, *, M, C):
    body = functools.partial(_final_body, M=M, C=C)
    full = lambda shape: pl.BlockSpec(shape, lambda: tuple(0 for _ in shape))
    return pl.pallas_call(
        body,
        in_specs=[full((M, 128)), full((128, C)), full((M, C)), full((M, C)),
                  full((M, C)), full((M, C)), full((1, C)), full((1, C))],
        out_specs=full((M, C)),
        out_shape=jax.ShapeDtypeStruct((M, C), jnp.float32),
    )(np_pad, wp_pad, gsum, gsq, gmx, gmn, g2, b2)


def kernel(p, x, o, W, gamma, beta):
    N = p.shape[0]
    seg = o.shape[0]
    n = N // seg
    m = n // STRIDE
    M = seg * m
    C = OUT_PLANES
    R = n // 128
    QR = m // 128
    QB = 128

    pseg = p.reshape(seg, n, 3)
    px = pseg[..., 0].reshape(seg, R, 128)
    py = pseg[..., 1].reshape(seg, R, 128)
    pz = pseg[..., 2].reshape(seg, R, 128)

    feats = jnp.concatenate(
        [p, x, jnp.zeros((N, 125 - IN_PLANES), jnp.float32)], axis=1)
    featsr = feats.reshape(seg, n, 128)
    wt = jnp.zeros((128, 128), jnp.float32).at[:3 + IN_PLANES, :C].set(W.T)

    fidx, npx, npy, npz, u = _run_fps(px, py, pz, featsr, wt,
                                      seg=seg, n=n, m=m, C=C)

    pxb = pseg[..., 0].reshape(seg, n // _CHW, _CHW).transpose(0, 2, 1)
    pyb = pseg[..., 1].reshape(seg, n // _CHW, _CHW).transpose(0, 2, 1)
    pzb = pseg[..., 2].reshape(seg, n // _CHW, _CHW).transpose(0, 2, 1)
    qx = npx.reshape(seg, m, 1)
    qy = npy.reshape(seg, m, 1)
    qz = npz.reshape(seg, m, 1)
    knn = _run_knn(pxb, pyb, pzb, qx, qy, qz, seg=seg, n=n, m=m, QB=QB)

    u_all = u.reshape(seg * n, 128)
    knn1d = knn.reshape(M * NSAMPLE)
    gsum, gsq, gmx, gmn = _run_gather_sc(u_all, knn1d,
                                         M=M, Ntot=seg * n, C=C)

    n_p = jnp.stack([npx.reshape(M), npy.reshape(M), npz.reshape(M)],
                    axis=-1)
    np_pad = jnp.concatenate([n_p, jnp.zeros((M, 125), jnp.float32)], axis=1)
    wp_pad = jnp.zeros((128, C), jnp.float32).at[:3].set(W[:, :3].T)
    out = _run_final(np_pad, wp_pad, gsum, gsq, gmx, gmn,
                     gamma.reshape(1, C), beta.reshape(1, C), M=M, C=C)

    return n_p, out, (o // STRIDE).astype(jnp.int32)


# speedup vs baseline: 1.6020x; 1.6020x over previous
import functools

import jax
import jax.numpy as jnp
from jax import lax
from jax.experimental import pallas as pl
from jax.experimental.pallas import tpu as pltpu

IN_PLANES = 64
OUT_PLANES = 64
STRIDE = 4
NSAMPLE = 16
EPS = 1e-5

_BIG_I32 = 2**30
_INF_F32 = 3e38


def _fps_body(px_ref, py_ref, pz_ref, feats_ref, wt_ref,
              fidx_ref, npx_ref, npy_ref, npz_ref, u_ref, *, seg, n, m, C):
    R = n // 128
    QR = m // 128
    px = px_ref[...]
    py = py_ref[...]
    pz = pz_ref[...]

    u_ref[...] = jnp.dot(
        feats_ref[...].reshape(seg * n, 128), wt_ref[...],
        preferred_element_type=jnp.float32).reshape(seg, n, 128)

    flat = jnp.broadcast_to(
        lax.broadcasted_iota(jnp.int32, (R, 128), 0) * 128
        + lax.broadcasted_iota(jnp.int32, (R, 128), 1), (seg, R, 128))
    qflat = jnp.broadcast_to(
        lax.broadcasted_iota(jnp.int32, (QR, 128), 0) * 128
        + lax.broadcasted_iota(jnp.int32, (QR, 128), 1), (seg, QR, 128))

    dist0 = jnp.full((seg, R, 128), 1e10, dtype=jnp.float32)
    lx0 = px[:, 0:1, 0:1]
    ly0 = py[:, 0:1, 0:1]
    lz0 = pz[:, 0:1, 0:1]
    idx0 = jnp.zeros((seg, QR, 128), dtype=jnp.int32)
    zero_q = jnp.zeros((seg, QR, 128), dtype=jnp.float32)
    sel0 = qflat == 0
    nx0 = jnp.where(sel0, lx0, zero_q)
    ny0 = jnp.where(sel0, ly0, zero_q)
    nz0 = jnp.where(sel0, lz0, zero_q)

    def body(i, st):
        dist, lx, ly, lz, idxv, nx, ny, nz = st
        dx = px - lx
        dy = py - ly
        dz = pz - lz
        d = dx * dx + dy * dy + dz * dz
        dist = jnp.minimum(dist, d)
        mx = jnp.max(dist, axis=(1, 2), keepdims=True)
        sel = jnp.min(jnp.where(dist == mx, flat, _BIG_I32),
                      axis=(1, 2), keepdims=True)
        hit = flat == sel
        sx = jnp.sum(jnp.where(hit, px, 0.0), axis=(1, 2), keepdims=True)
        sy = jnp.sum(jnp.where(hit, py, 0.0), axis=(1, 2), keepdims=True)
        sz = jnp.sum(jnp.where(hit, pz, 0.0), axis=(1, 2), keepdims=True)
        q = qflat == i
        idxv = jnp.where(q, sel, idxv)
        nx = jnp.where(q, sx, nx)
        ny = jnp.where(q, sy, ny)
        nz = jnp.where(q, sz, nz)
        return (dist, sx, sy, sz, idxv, nx, ny, nz)

    st = lax.fori_loop(1, m, body,
                       (dist0, lx0, ly0, lz0, idx0, nx0, ny0, nz0))
    _, _, _, _, idxv, nx, ny, nz = st
    fidx_ref[...] = idxv
    npx_ref[...] = nx
    npy_ref[...] = ny
    npz_ref[...] = nz


def _run_fps(px, py, pz, feats, wt, *, seg, n, m, C):
    R = n // 128
    QR = m // 128
    body = functools.partial(_fps_body, seg=seg, n=n, m=m, C=C)
    out_shapes = (
        jax.ShapeDtypeStruct((seg, QR, 128), jnp.int32),
        jax.ShapeDtypeStruct((seg, QR, 128), jnp.float32),
        jax.ShapeDtypeStruct((seg, QR, 128), jnp.float32),
        jax.ShapeDtypeStruct((seg, QR, 128), jnp.float32),
        jax.ShapeDtypeStruct((seg, n, 128), jnp.float32),
    )
    full = lambda shape: pl.BlockSpec(shape, lambda: tuple(0 for _ in shape))
    return pl.pallas_call(
        body,
        in_specs=[
            full((seg, R, 128)), full((seg, R, 128)), full((seg, R, 128)),
            full((seg, n, 128)), full((128, 128)),
        ],
        out_specs=(
            full((seg, QR, 128)), full((seg, QR, 128)), full((seg, QR, 128)),
            full((seg, QR, 128)), full((seg, n, 128)),
        ),
        out_shape=out_shapes,
    )(px, py, pz, feats, wt)


_NPRE = 4
_CHW = 64


def _knn_body(pxb_ref, pyb_ref, pzb_ref, qx_ref, qy_ref, qz_ref,
              knn_ref, *, n, QB):
    seg = pl.program_id(0)
    NCH = n // _CHW
    px = pxb_ref[0][None]
    py = pyb_ref[0][None]
    pz = pzb_ref[0][None]
    qx = qx_ref[0][:, :, None]
    qy = qy_ref[0][:, :, None]
    qz = qz_ref[0][:, :, None]
    dx = qx - px
    dy = qy - py
    dz = qz - pz
    d2 = dx * dx + dy * dy + dz * dz

    row = lax.broadcasted_iota(jnp.int32, (QB, _CHW, NCH), 1)
    chunk64 = lax.broadcasted_iota(jnp.int32, (QB, 1, NCH), 2) * _CHW
    cvs, cis = [], []
    for _ in range(_NPRE):
        mv = jnp.min(d2, axis=1, keepdims=True)
        am = jnp.min(jnp.where(d2 == mv, row, _BIG_I32),
                     axis=1, keepdims=True)
        cvs.append(mv[:, 0, :])
        cis.append((chunk64 + am)[:, 0, :])
        d2 = jnp.where(row == am, _INF_F32, d2)
    cv = jnp.concatenate(cvs, axis=1)
    ci = jnp.concatenate(cis, axis=1)

    laneNS = lax.broadcasted_iota(jnp.int32, (QB, NSAMPLE), 1)
    knn = jnp.zeros((QB, NSAMPLE), dtype=jnp.int32)
    for t in range(NSAMPLE):
        mv = jnp.min(cv, axis=1, keepdims=True)
        am = jnp.min(jnp.where(cv == mv, ci, _BIG_I32),
                     axis=1, keepdims=True)
        knn = jnp.where(laneNS == t, am, knn)
        cv = jnp.where((cv == mv) & (ci == am), _INF_F32, cv)
    knn_ref[0] = knn + seg * n


def _run_knn(pxb, pyb, pzb, qx, qy, qz, *, seg, n, m, QB):
    nqb = m // QB
    NCH = n // _CHW
    body = functools.partial(_knn_body, n=n, QB=QB)
    pspec = pl.BlockSpec((1, _CHW, NCH), lambda s, q: (s, 0, 0))
    qspec = pl.BlockSpec((1, QB, 1), lambda s, q: (s, q, 0))
    return pl.pallas_call(
        body,
        grid=(seg, nqb),
        in_specs=[pspec, pspec, pspec, qspec, qspec, qspec],
        out_specs=pl.BlockSpec((1, QB, NSAMPLE), lambda s, q: (s, q, 0)),
        out_shape=jax.ShapeDtypeStruct((seg, m, NSAMPLE), jnp.int32),
    )(pxb, pyb, pzb, qx, qy, qz)


def _gather_body(u_ref, knn_ref, gsum_ref, gsq_ref, gmx_ref, gmn_ref, *, QB, C):
    def body(q, carry):
        i0 = knn_ref[q, 0]
        r = u_ref[pl.ds(i0, 1), :]
        s_acc = r
        q_acc = r * r
        mx = r
        mn = r
        for t in range(1, NSAMPLE):
            i = knn_ref[q, t]
            r = u_ref[pl.ds(i, 1), :]
            s_acc = s_acc + r
            q_acc = q_acc + r * r
            mx = jnp.maximum(mx, r)
            mn = jnp.minimum(mn, r)
        gsum_ref[pl.ds(q, 1), :] = s_acc
        gsq_ref[pl.ds(q, 1), :] = q_acc
        gmx_ref[pl.ds(q, 1), :] = mx
        gmn_ref[pl.ds(q, 1), :] = mn
        return carry

    lax.fori_loop(0, QB, body, 0)


def _run_gather_tc(u_all, knn_flat, *, M, Ntot, C, QB):
    nb = M // QB
    body = functools.partial(_gather_body, QB=QB, C=C)
    out_sp = pl.BlockSpec((QB, C), lambda b: (b, 0))
    return pl.pallas_call(
        body,
        grid=(nb,),
        in_specs=[
            pl.BlockSpec((Ntot, C), lambda b: (0, 0)),
            pl.BlockSpec((QB, NSAMPLE), lambda b: (b, 0),
                         memory_space=pltpu.SMEM),
        ],
        out_specs=(out_sp, out_sp, out_sp, out_sp),
        out_shape=tuple(jax.ShapeDtypeStruct((M, C), jnp.float32)
                        for _ in range(4)),
    )(u_all, knn_flat)


def _run_gather_sc(u_all, knn1d, *, M, Ntot, C):
    import jax.experimental.pallas.tpu_sc as plsc
    info = plsc.get_sparse_core_info()
    NC, NS_SC = info.num_cores, info.num_subcores
    NW = NC * NS_SC
    qpw = M // NW
    CQ = 16
    nch = qpw // CQ
    ROWS = CQ * NSAMPLE
    mesh = plsc.VectorSubcoreMesh(core_axis_name="c", subcore_axis_name="s")

    @functools.partial(
        pl.kernel, mesh=mesh,
        out_type=tuple(jax.ShapeDtypeStruct((M, C), jnp.float32)
                       for _ in range(4)),
        scratch_types=[
            pltpu.VMEM((ROWS,), jnp.int32),
            pltpu.VMEM((ROWS, 128), jnp.float32),
            pltpu.VMEM((CQ, C), jnp.float32),
            pltpu.VMEM((CQ, C), jnp.float32),
            pltpu.VMEM((CQ, C), jnp.float32),
            pltpu.VMEM((CQ, C), jnp.float32),
            pltpu.SemaphoreType.DMA,
        ],
    )
    def body(u_hbm, knn_hbm, gsum_hbm, gsq_hbm, gmx_hbm, gmn_hbm,
             idx_v, rows_v, st_sum, st_sq, st_mx, st_mn, sem):
        wid = lax.axis_index("s") * NC + lax.axis_index("c")
        qbase0 = wid * qpw

        def chunk(t, carry):
            qbase = qbase0 + t * CQ
            pltpu.sync_copy(knn_hbm.at[pl.ds(qbase * NSAMPLE, ROWS)], idx_v)
            pltpu.async_copy(u_hbm.at[idx_v], rows_v, sem).wait()

            def per_q(q, c2):
                b = q * NSAMPLE
                for v in range(C // 16):
                    sl = pl.ds(v * 16, 16)
                    r = rows_v[b, sl]
                    ssum = r
                    ssq = r * r
                    smx = r
                    smn = r
                    for s in range(1, NSAMPLE):
                        r = rows_v[b + s, sl]
                        ssum = ssum + r
                        ssq = ssq + r * r
                        smx = jnp.maximum(smx, r)
                        smn = jnp.minimum(smn, r)
                    st_sum[q, sl] = ssum
                    st_sq[q, sl] = ssq
                    st_mx[q, sl] = smx
                    st_mn[q, sl] = smn
                return c2

            lax.fori_loop(0, CQ, per_q, 0)
            pltpu.sync_copy(st_sum, gsum_hbm.at[pl.ds(qbase, CQ)])
            pltpu.sync_copy(st_sq, gsq_hbm.at[pl.ds(qbase, CQ)])
            pltpu.sync_copy(st_mx, gmx_hbm.at[pl.ds(qbase, CQ)])
            pltpu.sync_copy(st_mn, gmn_hbm.at[pl.ds(qbase, CQ)])
            return carry

        lax.fori_loop(0, nch, chunk, 0)

    return body(u_all, knn1d)


def _final_body(np_ref, wp_ref, gsum_ref, gsq_ref, gmx_ref, gmn_ref,
                g_ref, b_ref, out_ref, *, M, C):
    zq = jnp.dot(np_ref[...], wp_ref[...], preferred_element_type=jnp.float32)
    gsum = gsum_ref[...]
    cnt = jnp.float32(M * NSAMPLE)
    s1 = (jnp.sum(gsum, axis=0, keepdims=True)
          - NSAMPLE * jnp.sum(zq, axis=0, keepdims=True))
    s2 = (jnp.sum(gsq_ref[...], axis=0, keepdims=True)
          - 2.0 * jnp.sum(zq * gsum, axis=0, keepdims=True)
          + NSAMPLE * jnp.sum(zq * zq, axis=0, keepdims=True))
    mean = s1 / cnt
    var = s2 / cnt - mean * mean
    a = g_ref[...] * lax.rsqrt(var + EPS)
    hsel = jnp.where(a >= 0.0, gmx_ref[...], gmn_ref[...]) - zq
    out_ref[...] = jnp.maximum(a * (hsel - mean) + b_ref[...], 0.0)


def _run_final(np_pad, wp_pad, gsum, gsq, gmx, gmn, g2, b2, *, M, C):
    body = functools.partial(_final_body, M=M, C=C)
    full = lambda shape: pl.BlockSpec(shape, lambda: tuple(0 for _ in shape))
    return pl.pallas_call(
        body,
        in_specs=[full((M, 128)), full((128, C)), full((M, C)), full((M, C)),
                  full((M, C)), full((M, C)), full((1, C)), full((1, C))],
        out_specs=full((M, C)),
        out_shape=jax.ShapeDtypeStruct((M, C), jnp.float32),
    )(np_pad, wp_pad, gsum, gsq, gmx, gmn, g2, b2)


def kernel(p, x, o, W, gamma, beta):
    N = p.shape[0]
    seg = o.shape[0]
    n = N // seg
    m = n // STRIDE
    M = seg * m
    C = OUT_PLANES
    R = n // 128
    QR = m // 128
    QB = 128

    pseg = p.reshape(seg, n, 3)
    px = pseg[..., 0].reshape(seg, R, 128)
    py = pseg[..., 1].reshape(seg, R, 128)
    pz = pseg[..., 2].reshape(seg, R, 128)

    feats = jnp.concatenate(
        [p, x, jnp.zeros((N, 125 - IN_PLANES), jnp.float32)], axis=1)
    featsr = feats.reshape(seg, n, 128)
    wt = jnp.zeros((128, 128), jnp.float32).at[:3 + IN_PLANES, :C].set(W.T)

    fidx, npx, npy, npz, u = _run_fps(px, py, pz, featsr, wt,
                                      seg=seg, n=n, m=m, C=C)

    pxb = pseg[..., 0].reshape(seg, n // _CHW, _CHW).transpose(0, 2, 1)
    pyb = pseg[..., 1].reshape(seg, n // _CHW, _CHW).transpose(0, 2, 1)
    pzb = pseg[..., 2].reshape(seg, n // _CHW, _CHW).transpose(0, 2, 1)
    qx = npx.reshape(seg, m, 1)
    qy = npy.reshape(seg, m, 1)
    qz = npz.reshape(seg, m, 1)
    knn = _run_knn(pxb, pyb, pzb, qx, qy, qz, seg=seg, n=n, m=m, QB=QB)

    u_all = u.reshape(seg * n, 128)
    knn1d = knn.reshape(M * NSAMPLE)
    gsum, gsq, gmx, gmn = _run_gather_sc(u_all, knn1d,
                                         M=M, Ntot=seg * n, C=C)

    n_p = jnp.stack([npx.reshape(M), npy.reshape(M), npz.reshape(M)],
                    axis=-1)
    np_pad = jnp.concatenate([n_p, jnp.zeros((M, 125), jnp.float32)], axis=1)
    wp_pad = jnp.zeros((128, C), jnp.float32).at[:3].set(W[:, :3].T)
    out = _run_final(np_pad, wp_pad, gsum, gsq, gmx, gmn,
                     gamma.reshape(1, C), beta.reshape(1, C), M=M, C=C)

    return n_p, out, (o // STRIDE).astype(jnp.int32)


# speedup vs baseline: 1.6915x; 1.0559x over previous
import functools

import jax
import jax.numpy as jnp
from jax import lax
from jax.experimental import pallas as pl
from jax.experimental.pallas import tpu as pltpu

IN_PLANES = 64
OUT_PLANES = 64
STRIDE = 4
NSAMPLE = 16
EPS = 1e-5

_BIG_I32 = 2**30
_INF_F32 = 3e38


def _fps_body(px_ref, py_ref, pz_ref, feats_ref, wt_ref,
              fidx_ref, npx_ref, npy_ref, npz_ref, u_ref, *, seg, n, m, C):
    R = n // 128
    QR = m // 128
    px = px_ref[...]
    py = py_ref[...]
    pz = pz_ref[...]

    u_ref[...] = jnp.dot(
        feats_ref[...].reshape(seg * n, 128), wt_ref[...],
        preferred_element_type=jnp.float32).reshape(seg, n, 128)

    flat = jnp.broadcast_to(
        lax.broadcasted_iota(jnp.int32, (R, 128), 0) * 128
        + lax.broadcasted_iota(jnp.int32, (R, 128), 1), (seg, R, 128))
    qflat = jnp.broadcast_to(
        lax.broadcasted_iota(jnp.int32, (QR, 128), 0) * 128
        + lax.broadcasted_iota(jnp.int32, (QR, 128), 1), (seg, QR, 128))

    dist0 = jnp.full((seg, R, 128), 1e10, dtype=jnp.float32)
    lx0 = px[:, 0:1, 0:1]
    ly0 = py[:, 0:1, 0:1]
    lz0 = pz[:, 0:1, 0:1]
    idx0 = jnp.zeros((seg, QR, 128), dtype=jnp.int32)
    zero_q = jnp.zeros((seg, QR, 128), dtype=jnp.float32)
    sel0 = qflat == 0
    nx0 = jnp.where(sel0, lx0, zero_q)
    ny0 = jnp.where(sel0, ly0, zero_q)
    nz0 = jnp.where(sel0, lz0, zero_q)

    def body(i, st):
        dist, lx, ly, lz, idxv, nx, ny, nz = st
        dx = px - lx
        dy = py - ly
        dz = pz - lz
        d = dx * dx + dy * dy + dz * dz
        dist = jnp.minimum(dist, d)
        mx = jnp.max(dist, axis=(1, 2), keepdims=True)
        sel = jnp.min(jnp.where(dist == mx, flat, _BIG_I32),
                      axis=(1, 2), keepdims=True)
        hit = flat == sel
        sx = jnp.sum(jnp.where(hit, px, 0.0), axis=(1, 2), keepdims=True)
        sy = jnp.sum(jnp.where(hit, py, 0.0), axis=(1, 2), keepdims=True)
        sz = jnp.sum(jnp.where(hit, pz, 0.0), axis=(1, 2), keepdims=True)
        q = qflat == i
        idxv = jnp.where(q, sel, idxv)
        nx = jnp.where(q, sx, nx)
        ny = jnp.where(q, sy, ny)
        nz = jnp.where(q, sz, nz)
        return (dist, sx, sy, sz, idxv, nx, ny, nz)

    st = lax.fori_loop(1, m, body,
                       (dist0, lx0, ly0, lz0, idx0, nx0, ny0, nz0),
                       unroll=4)
    _, _, _, _, idxv, nx, ny, nz = st
    fidx_ref[...] = idxv
    npx_ref[...] = nx
    npy_ref[...] = ny
    npz_ref[...] = nz


def _run_fps(px, py, pz, feats, wt, *, seg, n, m, C):
    R = n // 128
    QR = m // 128
    body = functools.partial(_fps_body, seg=seg, n=n, m=m, C=C)
    out_shapes = (
        jax.ShapeDtypeStruct((seg, QR, 128), jnp.int32),
        jax.ShapeDtypeStruct((seg, QR, 128), jnp.float32),
        jax.ShapeDtypeStruct((seg, QR, 128), jnp.float32),
        jax.ShapeDtypeStruct((seg, QR, 128), jnp.float32),
        jax.ShapeDtypeStruct((seg, n, 128), jnp.float32),
    )
    full = lambda shape: pl.BlockSpec(shape, lambda: tuple(0 for _ in shape))
    return pl.pallas_call(
        body,
        in_specs=[
            full((seg, R, 128)), full((seg, R, 128)), full((seg, R, 128)),
            full((seg, n, 128)), full((128, 128)),
        ],
        out_specs=(
            full((seg, QR, 128)), full((seg, QR, 128)), full((seg, QR, 128)),
            full((seg, QR, 128)), full((seg, n, 128)),
        ),
        out_shape=out_shapes,
    )(px, py, pz, feats, wt)


_NPRE = 4
_CHW = 64


def _knn_body(pxb_ref, pyb_ref, pzb_ref, qx_ref, qy_ref, qz_ref,
              knn_ref, *, n, QB):
    seg = pl.program_id(0)
    NCH = n // _CHW
    px = pxb_ref[0][None]
    py = pyb_ref[0][None]
    pz = pzb_ref[0][None]
    qx = qx_ref[0][:, :, None]
    qy = qy_ref[0][:, :, None]
    qz = qz_ref[0][:, :, None]
    dx = qx - px
    dy = qy - py
    dz = qz - pz
    d2 = dx * dx + dy * dy + dz * dz

    row = lax.broadcasted_iota(jnp.int32, (QB, _CHW, NCH), 1)
    chunk64 = lax.broadcasted_iota(jnp.int32, (QB, 1, NCH), 2) * _CHW
    cvs, cis = [], []
    for _ in range(_NPRE):
        mv = jnp.min(d2, axis=1, keepdims=True)
        am = jnp.min(jnp.where(d2 == mv, row, _BIG_I32),
                     axis=1, keepdims=True)
        cvs.append(mv[:, 0, :])
        cis.append((chunk64 + am)[:, 0, :])
        d2 = jnp.where(row == am, _INF_F32, d2)
    cv = jnp.concatenate(cvs, axis=1)
    ci = jnp.concatenate(cis, axis=1)

    laneNS = lax.broadcasted_iota(jnp.int32, (QB, NSAMPLE), 1)
    knn = jnp.zeros((QB, NSAMPLE), dtype=jnp.int32)
    for t in range(NSAMPLE):
        mv = jnp.min(cv, axis=1, keepdims=True)
        am = jnp.min(jnp.where(cv == mv, ci, _BIG_I32),
                     axis=1, keepdims=True)
        knn = jnp.where(laneNS == t, am, knn)
        cv = jnp.where((cv == mv) & (ci == am), _INF_F32, cv)
    knn_ref[0] = knn + seg * n


def _run_knn(pxb, pyb, pzb, qx, qy, qz, *, seg, n, m, QB):
    nqb = m // QB
    NCH = n // _CHW
    body = functools.partial(_knn_body, n=n, QB=QB)
    pspec = pl.BlockSpec((1, _CHW, NCH), lambda s, q: (s, 0, 0))
    qspec = pl.BlockSpec((1, QB, 1), lambda s, q: (s, q, 0))
    return pl.pallas_call(
        body,
        grid=(seg, nqb),
        in_specs=[pspec, pspec, pspec, qspec, qspec, qspec],
        out_specs=pl.BlockSpec((1, QB, NSAMPLE), lambda s, q: (s, q, 0)),
        out_shape=jax.ShapeDtypeStruct((seg, m, NSAMPLE), jnp.int32),
    )(pxb, pyb, pzb, qx, qy, qz)


def _gather_body(u_ref, knn_ref, gsum_ref, gsq_ref, gmx_ref, gmn_ref, *, QB, C):
    def body(q, carry):
        i0 = knn_ref[q, 0]
        r = u_ref[pl.ds(i0, 1), :]
        s_acc = r
        q_acc = r * r
        mx = r
        mn = r
        for t in range(1, NSAMPLE):
            i = knn_ref[q, t]
            r = u_ref[pl.ds(i, 1), :]
            s_acc = s_acc + r
            q_acc = q_acc + r * r
            mx = jnp.maximum(mx, r)
            mn = jnp.minimum(mn, r)
        gsum_ref[pl.ds(q, 1), :] = s_acc
        gsq_ref[pl.ds(q, 1), :] = q_acc
        gmx_ref[pl.ds(q, 1), :] = mx
        gmn_ref[pl.ds(q, 1), :] = mn
        return carry

    lax.fori_loop(0, QB, body, 0)


def _run_gather_tc(u_all, knn_flat, *, M, Ntot, C, QB):
    nb = M // QB
    body = functools.partial(_gather_body, QB=QB, C=C)
    out_sp = pl.BlockSpec((QB, C), lambda b: (b, 0))
    return pl.pallas_call(
        body,
        grid=(nb,),
        in_specs=[
            pl.BlockSpec((Ntot, C), lambda b: (0, 0)),
            pl.BlockSpec((QB, NSAMPLE), lambda b: (b, 0),
                         memory_space=pltpu.SMEM),
        ],
        out_specs=(out_sp, out_sp, out_sp, out_sp),
        out_shape=tuple(jax.ShapeDtypeStruct((M, C), jnp.float32)
                        for _ in range(4)),
    )(u_all, knn_flat)


def _run_gather_sc(u_all, knn1d, *, M, Ntot, C):
    import jax.experimental.pallas.tpu_sc as plsc
    info = plsc.get_sparse_core_info()
    NC, NS_SC = info.num_cores, info.num_subcores
    NW = NC * NS_SC
    qpw = M // NW
    CQ = 16
    nch = qpw // CQ
    ROWS = CQ * NSAMPLE
    mesh = plsc.VectorSubcoreMesh(core_axis_name="c", subcore_axis_name="s")

    @functools.partial(
        pl.kernel, mesh=mesh,
        out_type=tuple(jax.ShapeDtypeStruct((M, C), jnp.float32)
                       for _ in range(4)),
        scratch_types=[
            pltpu.VMEM((ROWS,), jnp.int32),
            pltpu.VMEM((ROWS, 128), jnp.float32),
            pltpu.VMEM((CQ, C), jnp.float32),
            pltpu.VMEM((CQ, C), jnp.float32),
            pltpu.VMEM((CQ, C), jnp.float32),
            pltpu.VMEM((CQ, C), jnp.float32),
            pltpu.SemaphoreType.DMA,
        ],
    )
    def body(u_hbm, knn_hbm, gsum_hbm, gsq_hbm, gmx_hbm, gmn_hbm,
             idx_v, rows_v, st_sum, st_sq, st_mx, st_mn, sem):
        wid = lax.axis_index("s") * NC + lax.axis_index("c")
        qbase0 = wid * qpw

        def chunk(t, carry):
            qbase = qbase0 + t * CQ
            pltpu.sync_copy(knn_hbm.at[pl.ds(qbase * NSAMPLE, ROWS)], idx_v)
            pltpu.async_copy(u_hbm.at[idx_v], rows_v, sem).wait()

            def per_q(q, c2):
                b = q * NSAMPLE
                for v in range(C // 16):
                    sl = pl.ds(v * 16, 16)
                    r = rows_v[b, sl]
                    ssum = r
                    ssq = r * r
                    smx = r
                    smn = r
                    for s in range(1, NSAMPLE):
                        r = rows_v[b + s, sl]
                        ssum = ssum + r
                        ssq = ssq + r * r
                        smx = jnp.maximum(smx, r)
                        smn = jnp.minimum(smn, r)
                    st_sum[q, sl] = ssum
                    st_sq[q, sl] = ssq
                    st_mx[q, sl] = smx
                    st_mn[q, sl] = smn
                return c2

            lax.fori_loop(0, CQ, per_q, 0)
            pltpu.sync_copy(st_sum, gsum_hbm.at[pl.ds(qbase, CQ)])
            pltpu.sync_copy(st_sq, gsq_hbm.at[pl.ds(qbase, CQ)])
            pltpu.sync_copy(st_mx, gmx_hbm.at[pl.ds(qbase, CQ)])
            pltpu.sync_copy(st_mn, gmn_hbm.at[pl.ds(qbase, CQ)])
            return carry

        lax.fori_loop(0, nch, chunk, 0)

    return body(u_all, knn1d)


def _final_body(np_ref, wp_ref, gsum_ref, gsq_ref, gmx_ref, gmn_ref,
                g_ref, b_ref, out_ref, *, M, C):
    zq = jnp.dot(np_ref[...], wp_ref[...], preferred_element_type=jnp.float32)
    gsum = gsum_ref[...]
    cnt = jnp.float32(M * NSAMPLE)
    s1 = (jnp.sum(gsum, axis=0, keepdims=True)
          - NSAMPLE * jnp.sum(zq, axis=0, keepdims=True))
    s2 = (jnp.sum(gsq_ref[...], axis=0, keepdims=True)
          - 2.0 * jnp.sum(zq * gsum, axis=0, keepdims=True)
          + NSAMPLE * jnp.sum(zq * zq, axis=0, keepdims=True))
    mean = s1 / cnt
    var = s2 / cnt - mean * mean
    a = g_ref[...] * lax.rsqrt(var + EPS)
    hsel = jnp.where(a >= 0.0, gmx_ref[...], gmn_ref[...]) - zq
    out_ref[...] = jnp.maximum(a * (hsel - mean) + b_ref[...], 0.0)


def _run_final(np_pad, wp_pad, gsum, gsq, gmx, gmn, g2, b2, *, M, C):
    body = functools.partial(_final_body, M=M, C=C)
    full = lambda shape: pl.BlockSpec(shape, lambda: tuple(0 for _ in shape))
    return pl.pallas_call(
        body,
        in_specs=[full((M, 128)), full((128, C)), full((M, C)), full((M, C)),
                  full((M, C)), full((M, C)), full((1, C)), full((1, C))],
        out_specs=full((M, C)),
        out_shape=jax.ShapeDtypeStruct((M, C), jnp.float32),
    )(np_pad, wp_pad, gsum, gsq, gmx, gmn, g2, b2)


def kernel(p, x, o, W, gamma, beta):
    N = p.shape[0]
    seg = o.shape[0]
    n = N // seg
    m = n // STRIDE
    M = seg * m
    C = OUT_PLANES
    R = n // 128
    QR = m // 128
    QB = 128

    pseg = p.reshape(seg, n, 3)
    px = pseg[..., 0].reshape(seg, R, 128)
    py = pseg[..., 1].reshape(seg, R, 128)
    pz = pseg[..., 2].reshape(seg, R, 128)

    feats = jnp.concatenate(
        [p, x, jnp.zeros((N, 125 - IN_PLANES), jnp.float32)], axis=1)
    featsr = feats.reshape(seg, n, 128)
    wt = jnp.zeros((128, 128), jnp.float32).at[:3 + IN_PLANES, :C].set(W.T)

    fidx, npx, npy, npz, u = _run_fps(px, py, pz, featsr, wt,
                                      seg=seg, n=n, m=m, C=C)

    pxb = pseg[..., 0].reshape(seg, n // _CHW, _CHW).transpose(0, 2, 1)
    pyb = pseg[..., 1].reshape(seg, n // _CHW, _CHW).transpose(0, 2, 1)
    pzb = pseg[..., 2].reshape(seg, n // _CHW, _CHW).transpose(0, 2, 1)
    qx = npx.reshape(seg, m, 1)
    qy = npy.reshape(seg, m, 1)
    qz = npz.reshape(seg, m, 1)
    knn = _run_knn(pxb, pyb, pzb, qx, qy, qz, seg=seg, n=n, m=m, QB=QB)

    u_all = u.reshape(seg * n, 128)
    knn1d = knn.reshape(M * NSAMPLE)
    gsum, gsq, gmx, gmn = _run_gather_sc(u_all, knn1d,
                                         M=M, Ntot=seg * n, C=C)

    n_p = jnp.stack([npx.reshape(M), npy.reshape(M), npz.reshape(M)],
                    axis=-1)
    np_pad = jnp.concatenate([n_p, jnp.zeros((M, 125), jnp.float32)], axis=1)
    wp_pad = jnp.zeros((128, C), jnp.float32).at[:3].set(W[:, :3].T)
    out = _run_final(np_pad, wp_pad, gsum, gsq, gmx, gmn,
                     gamma.reshape(1, C), beta.reshape(1, C), M=M, C=C)

    return n_p, out, (o // STRIDE).astype(jnp.int32)
